# baseline (speedup 1.0000x reference)
import functools

import jax
import jax.numpy as jnp
from jax import lax
from jax.experimental import pallas as pl
from jax.experimental.pallas import tpu as pltpu
from jax.experimental.pallas import tpu_sc as plsc

N_NODES = 10000
N_EDGES = 320000
HIDDEN = 128
L = 16
NC, NS = 2, 16
NW = NC * NS
E_PER_W = N_EDGES // NW
B = 80
NCHUNK = E_PER_W // B
GROUPS = B // L
TM = 400
GW = N_NODES * N_NODES // 2

_mesh = plsc.VectorSubcoreMesh(
    core_axis_name="c", subcore_axis_name="s", num_cores=NC, num_subcores=NS
)


def _mm_body(a_ref, b_ref, o_ref):
    o_ref[...] = lax.dot_general(
        a_ref[...], b_ref[...], (((1,), (1,)), ((), ())),
        preferred_element_type=jnp.float32,
    ).astype(jnp.bfloat16)


_gram = pl.pallas_call(
    _mm_body,
    grid=(N_NODES // TM,),
    in_specs=[
        pl.BlockSpec((TM, HIDDEN), lambda i: (i, 0)),
        pl.BlockSpec((N_NODES, HIDDEN), lambda i: (0, 0)),
    ],
    out_specs=pl.BlockSpec((TM, N_NODES), lambda i: (i, 0)),
    out_shape=jax.ShapeDtypeStruct((N_NODES, N_NODES), jnp.bfloat16),
)


@functools.partial(
    pl.kernel,
    out_type=jax.ShapeDtypeStruct((NW, NCHUNK, B), jnp.float32),
    mesh=_mesh,
    scratch_types=[
        pltpu.VMEM((NCHUNK, B), jnp.int32),
        pltpu.VMEM((NCHUNK, B), jnp.int32),
        pltpu.VMEM((B,), jnp.int32),
        pltpu.VMEM((B,), jnp.int32),
        pltpu.VMEM((B,), jnp.int32),
        pltpu.VMEM((B,), jnp.int32),
        pltpu.VMEM((B,), jnp.int32),
        pltpu.VMEM((B,), jnp.int32),
        pltpu.VMEM((NCHUNK, B), jnp.float32),
        pltpu.SemaphoreType.DMA,
        pltpu.SemaphoreType.DMA,
    ],
    compiler_params=pltpu.CompilerParams(
        needs_layout_passes=False, use_tc_tiling_on_sc=False),
)
def _edge_lookup(row_hbm, col_hbm, gw_hbm, out_hbm,
                 ridx_v, cidx_v, wi_a, wi_b, pa_a, pa_b, gw_a, gw_b,
                 out_v, sem_a, sem_b):
    wid = lax.axis_index("s") * NC + lax.axis_index("c")

    pltpu.sync_copy(row_hbm.at[wid], ridx_v)
    pltpu.sync_copy(col_hbm.at[wid], cidx_v)

    def prep_issue(ci, wi, pa, gw, sem):
        for g in range(GROUPS):
            r = ridx_v[ci, pl.ds(g * L, L)]
            c = cidx_v[ci, pl.ds(g * L, L)]
            flat = r * N_NODES + c
            wi[pl.ds(g * L, L)] = lax.shift_right_logical(flat, 1)
            pa[pl.ds(g * L, L)] = lax.bitwise_and(flat, 1)
        pltpu.async_copy(gw_hbm.at[wi], gw, sem)

    def wait(wi, gw, sem):
        pltpu.make_async_copy(gw_hbm.at[wi], gw, sem).wait()

    def consume(ci, pa, gw):
        for g in range(GROUPS):
            w = plsc.bitcast(gw[pl.ds(g * L, L)], jnp.bfloat16)
            p0, p1 = plsc.unpack(w, format=plsc.PackFormat.INTERLEAVED)
            pr = pa[pl.ds(g * L, L)]
            v = jnp.where(pr == 0, p0, p1)
            out_v[ci, pl.ds(g * L, L)] = 1.0 / (1.0 + jnp.exp(-v))

    prep_issue(0, wi_a, pa_a, gw_a, sem_a)

    def pair_body(k, _):
        c0 = 2 * k
        prep_issue(c0 + 1, wi_b, pa_b, gw_b, sem_b)
        wait(wi_a, gw_a, sem_a)
        consume(c0, pa_a, gw_a)
        prep_issue(c0 + 2, wi_a, pa_a, gw_a, sem_a)
        wait(wi_b, gw_b, sem_b)
        consume(c0 + 1, pa_b, gw_b)
        return 0

    lax.fori_loop(0, NCHUNK // 2, pair_body, 0)
    wait(wi_a, gw_a, sem_a)
    consume(NCHUNK - 1, pa_a, gw_a)

    pltpu.sync_copy(out_v, out_hbm.at[wid])


def kernel(z, edge_index):
    zb = z.astype(jnp.bfloat16)
    g = _gram(zb, zb)
    gwords = jax.lax.bitcast_convert_type(
        g.reshape(N_NODES * N_NODES // 2, 2), jnp.int32)
    row = edge_index[0].reshape(NW, NCHUNK, B)
    col = edge_index[1].reshape(NW, NCHUNK, B)
    out = _edge_lookup(row, col, gwords)
    return out.reshape(N_EDGES)


# speedup vs baseline: 45.0733x; 45.0733x over previous
import functools

import jax
import jax.numpy as jnp
from jax import lax
from jax.experimental import pallas as pl
from jax.experimental.pallas import tpu as pltpu
from jax.experimental.pallas import tpu_sc as plsc

N_NODES = 10000
N_EDGES = 320000
HIDDEN = 128
L = 16
NC, NS = 2, 16
NW = NC * NS
E_PER_W = N_EDGES // NW
B = 80
NCHUNK = E_PER_W // B
GROUPS = B // L
TM = 400
GW = N_NODES * N_NODES // 2

_mesh = plsc.VectorSubcoreMesh(
    core_axis_name="c", subcore_axis_name="s", num_cores=NC, num_subcores=NS
)


def _mm_body(a_ref, b_ref, o_ref):
    o_ref[...] = lax.dot_general(
        a_ref[...], b_ref[...], (((1,), (1,)), ((), ())),
        preferred_element_type=jnp.float32,
    )


_gram = pl.pallas_call(
    _mm_body,
    grid=(N_NODES // TM,),
    in_specs=[
        pl.BlockSpec((TM, HIDDEN), lambda i: (i, 0)),
        pl.BlockSpec((N_NODES, HIDDEN), lambda i: (0, 0)),
    ],
    out_specs=pl.BlockSpec((TM, N_NODES), lambda i: (i, 0)),
    out_shape=jax.ShapeDtypeStruct((N_NODES, N_NODES), jnp.float32),
)


@functools.partial(
    pl.kernel,
    out_type=jax.ShapeDtypeStruct((NW, NCHUNK, B), jnp.float32),
    mesh=_mesh,
    scratch_types=[
        pltpu.VMEM((NCHUNK, B), jnp.int32),
        pltpu.VMEM((NCHUNK, B), jnp.int32),
        pltpu.VMEM((B,), jnp.int32),
        pltpu.VMEM((B,), jnp.int32),
        pltpu.VMEM((B,), jnp.float32),
        pltpu.VMEM((B,), jnp.float32),
        pltpu.VMEM((NCHUNK, B), jnp.float32),
        pltpu.SemaphoreType.DMA,
        pltpu.SemaphoreType.DMA,
    ],
    compiler_params=pltpu.CompilerParams(
        needs_layout_passes=False, use_tc_tiling_on_sc=False),
)
def _edge_lookup(row_hbm, col_hbm, gw_hbm, out_hbm,
                 ridx_v, cidx_v, wi_a, wi_b, gw_a, gw_b,
                 out_v, sem_a, sem_b):
    wid = lax.axis_index("s") * NC + lax.axis_index("c")

    pltpu.sync_copy(row_hbm.at[wid], ridx_v)
    pltpu.sync_copy(col_hbm.at[wid], cidx_v)

    def prep_issue(ci, wi, gw, sem):
        for g in range(GROUPS):
            r = ridx_v[ci, pl.ds(g * L, L)]
            c = cidx_v[ci, pl.ds(g * L, L)]
            wi[pl.ds(g * L, L)] = r * N_NODES + c
        pltpu.async_copy(gw_hbm.at[wi], gw, sem)

    def wait(wi, gw, sem):
        pltpu.make_async_copy(gw_hbm.at[wi], gw, sem).wait()

    def consume(ci, gw):
        for g in range(GROUPS):
            v = gw[pl.ds(g * L, L)]
            out_v[ci, pl.ds(g * L, L)] = 1.0 / (1.0 + jnp.exp(-v))

    prep_issue(0, wi_a, gw_a, sem_a)

    def pair_body(k, _):
        c0 = 2 * k
        prep_issue(c0 + 1, wi_b, gw_b, sem_b)
        wait(wi_a, gw_a, sem_a)
        consume(c0, gw_a)
        prep_issue(c0 + 2, wi_a, gw_a, sem_a)
        wait(wi_b, gw_b, sem_b)
        consume(c0 + 1, gw_b)
        return 0

    lax.fori_loop(0, NCHUNK // 2, pair_body, 0)
    wait(wi_a, gw_a, sem_a)
    consume(NCHUNK - 1, gw_a)

    pltpu.sync_copy(out_v, out_hbm.at[wid])


def kernel(z, edge_index):
    zb = z.astype(jnp.bfloat16)
    gwords = _gram(zb, zb).reshape(N_NODES * N_NODES)
    row = edge_index[0].reshape(NW, NCHUNK, B)
    col = edge_index[1].reshape(NW, NCHUNK, B)
    out = _edge_lookup(row, col, gwords)
    return out.reshape(N_EDGES)
